# Initial kernel scaffold; baseline (speedup 1.0000x reference)
import functools

import jax
import jax.numpy as jnp
from jax import lax
from jax.experimental import pallas as pl
from jax.experimental.pallas import tpu as pltpu
from jax.experimental.pallas import tpu_sc as plsc

_L = 16
_NW = 32
_C = 128


def kernel(x, weights, table):
    B, F = x.shape
    V, D = table.shape
    n_total = B * F
    per_w = n_total // _NW
    n_chunks = per_w // _C
    assert n_total % _NW == 0 and per_w % _C == 0 and D % _L == 0

    x_split = x.reshape(_NW, n_chunks, _C)
    w_pad = jnp.zeros((_L,), jnp.float32).at[: weights.shape[0]].set(weights)

    mesh = plsc.VectorSubcoreMesh(core_axis_name="c", subcore_axis_name="s")

    @functools.partial(
        pl.kernel,
        mesh=mesh,
        out_type=jax.ShapeDtypeStruct((n_total, D), jnp.float32),
        scratch_types=[
            pltpu.VMEM((n_chunks, _C), jnp.int32),
            pltpu.VMEM((_L,), jnp.float32),
            pltpu.VMEM((_C, D), jnp.float32),
            pltpu.SemaphoreType.DMA,
        ],
    )
    def run(x_hbm, w_hbm, table_hbm, out_hbm, idx_v, w_v, rows_v, sem):
        wid = lax.axis_index("s") * 2 + lax.axis_index("c")
        base = wid * per_w

        pltpu.sync_copy(w_hbm, w_v)
        pltpu.sync_copy(x_hbm.at[wid], idx_v)

        def bcast(k):
            return plsc.load_gather(w_v, [jnp.full((_L,), k, jnp.int32)])

        w0, w1, w2 = bcast(0), bcast(1), bcast(2)
        s2 = w2
        s1 = w1 + s2
        s0 = w0 + s1
        scales = [s0, s0, s1, s1, s2, s2, s2, s2]

        for j in range(n_chunks):
            pltpu.async_copy(table_hbm.at[idx_v.at[j]], rows_v, sem).wait()

            def row_body(i, _):
                for jc in range(D // _L):
                    sl = pl.ds(jc * _L, _L)
                    rows_v[i, sl] = rows_v[i, sl] * scales[jc]
                return 0

            lax.fori_loop(0, _C, row_body, 0)
            pltpu.sync_copy(rows_v, out_hbm.at[pl.ds(base + j * _C, _C)])

    out_flat = run(x_split, w_pad, table)
    return out_flat.reshape(B, F, D)


# speedup vs baseline: 1.6812x; 1.6812x over previous
import functools

import jax
import jax.numpy as jnp
from jax import lax
from jax.experimental import pallas as pl
from jax.experimental.pallas import tpu as pltpu
from jax.experimental.pallas import tpu_sc as plsc

_L = 16
_NW = 32
_C = 128


def kernel(x, weights, table):
    B, F = x.shape
    V, D = table.shape
    n_total = B * F
    per_w = n_total // _NW
    n_chunks = per_w // _C
    assert n_total % _NW == 0 and per_w % _C == 0 and D % _L == 0

    x_split = x.reshape(_NW, n_chunks, _C)
    w_pad = jnp.zeros((_L,), jnp.float32).at[: weights.shape[0]].set(weights)

    mesh = plsc.VectorSubcoreMesh(core_axis_name="c", subcore_axis_name="s")

    @functools.partial(
        pl.kernel,
        mesh=mesh,
        out_type=jax.ShapeDtypeStruct((n_total, D), jnp.float32),
        scratch_types=[
            pltpu.VMEM((n_chunks, _C), jnp.int32),
            pltpu.VMEM((_L,), jnp.float32),
            pltpu.VMEM((_C, D), jnp.float32),
            pltpu.SemaphoreType.DMA,
        ],
    )
    def run(x_hbm, w_hbm, table_hbm, out_hbm, idx_v, w_v, rows_v, sem):
        wid = lax.axis_index("s") * 2 + lax.axis_index("c")
        base = wid * per_w

        pltpu.sync_copy(w_hbm, w_v)
        pltpu.sync_copy(x_hbm.at[wid], idx_v)

        ones = jnp.ones((_L,), jnp.float32)
        w_vec = w_v[...]
        w0, w1, w2 = w_vec[0] * ones, w_vec[1] * ones, w_vec[2] * ones
        s2 = w2
        s1 = w1 + s2
        s0 = w0 + s1
        scales = [s0, s0, s1, s1, s2, s2, s2, s2]

        for j in range(n_chunks):
            pltpu.async_copy(table_hbm.at[idx_v.at[j]], rows_v, sem).wait()

            def row_body(i, _):
                for jc in range(D // _L):
                    sl = pl.ds(jc * _L, _L)
                    rows_v[i, sl] = rows_v[i, sl] * scales[jc]
                return 0

            lax.fori_loop(0, _C, row_body, 0)
            pltpu.sync_copy(rows_v, out_hbm.at[pl.ds(base + j * _C, _C)])

    out_flat = run(x_split, w_pad, table)
    return out_flat.reshape(B, F, D)


# speedup vs baseline: 1.9592x; 1.1653x over previous
import functools

import jax
import jax.numpy as jnp
from jax import lax
from jax.experimental import pallas as pl
from jax.experimental.pallas import tpu as pltpu
from jax.experimental.pallas import tpu_sc as plsc

_L = 16
_NW = 32
_C = 128


def kernel(x, weights, table):
    B, F = x.shape
    V, D = table.shape
    n_total = B * F
    per_w = n_total // _NW
    n_chunks = per_w // _C
    assert n_total % _NW == 0 and per_w % _C == 0 and D % _L == 0

    x_split = x.reshape(_NW, n_chunks, _C)
    w_pad = jnp.zeros((_L,), jnp.float32).at[: weights.shape[0]].set(weights)

    mesh = plsc.VectorSubcoreMesh(core_axis_name="c", subcore_axis_name="s")

    @functools.partial(
        pl.kernel,
        mesh=mesh,
        out_type=jax.ShapeDtypeStruct((n_total, D), jnp.float32),
        scratch_types=[
            pltpu.VMEM((n_chunks, _C), jnp.int32),
            pltpu.VMEM((_L,), jnp.float32),
            pltpu.VMEM((2, _C, D), jnp.float32),
            pltpu.SemaphoreType.DMA,
            pltpu.SemaphoreType.DMA,
            pltpu.SemaphoreType.DMA,
            pltpu.SemaphoreType.DMA,
        ],
    )
    def run(x_hbm, w_hbm, table_hbm, out_hbm, idx_v, w_v, rows_v, g0, g1, p0, p1):
        wid = lax.axis_index("s") * 2 + lax.axis_index("c")
        base = wid * per_w

        pltpu.sync_copy(w_hbm, w_v)
        pltpu.sync_copy(x_hbm.at[wid], idx_v)

        ones = jnp.ones((_L,), jnp.float32)
        w_vec = w_v[...]
        w0, w1, w2 = w_vec[0] * ones, w_vec[1] * ones, w_vec[2] * ones
        s2 = w2
        s1 = w1 + s2
        s0 = w0 + s1
        scales = [s0, s0, s1, s1, s2, s2, s2, s2]

        gsem = [g0, g1]
        wsem = [p0, p1]

        def gather(j):
            b = j % 2
            return pltpu.async_copy(table_hbm.at[idx_v.at[j]], rows_v.at[b], gsem[b])

        def write(j):
            b = j % 2
            return pltpu.async_copy(
                rows_v.at[b], out_hbm.at[pl.ds(base + j * _C, _C)], wsem[b]
            )

        def scale_buf(b):
            def row_body(i, _):
                for jc in range(D // _L):
                    sl = pl.ds(jc * _L, _L)
                    rows_v[b, i, sl] = rows_v[b, i, sl] * scales[jc]
                return 0

            lax.fori_loop(0, _C, row_body, 0)

        pend_w = [None, None]
        gh = [None, None]
        gh[0] = gather(0)
        for j in range(n_chunks):
            b = j % 2
            nb = (j + 1) % 2
            if j + 1 < n_chunks:
                if pend_w[nb] is not None:
                    pend_w[nb].wait()
                    pend_w[nb] = None
                gh[nb] = gather(j + 1)
            gh[b].wait()
            scale_buf(b)
            pend_w[b] = write(j)
        for b in range(2):
            if pend_w[b] is not None:
                pend_w[b].wait()

    out_flat = run(x_split, w_pad, table)
    return out_flat.reshape(B, F, D)
